# Initial kernel scaffold; baseline (speedup 1.0000x reference)
import jax
import jax.numpy as jnp
from jax.experimental import pallas as pl


def kernel(x, edge_index, batch, W_in, b_in, conv_W, conv_b, gn_w, gn_b, gn_ms, Wc1, bc1, Wc2, bc2, Wc3, bc3, Ws1, bs1, Ws2, bs2):
    raise NotImplementedError("write your pallas kernel here")


# speedup vs baseline: 9.6381x; 9.6381x over previous
import functools

import jax
import jax.numpy as jnp
from jax import lax
from jax.experimental import pallas as pl
from jax.experimental.pallas import tpu as pltpu
from jax.experimental.pallas import tpu_sc as plsc

N = 50000
E = 800000
D = 128
H = 64
NP = 50048
HALF = NP // 2
ACC_ROWS = HALF + 64
TPR = HALF // 16
CH = 128
EC = E // CH
DEG_TPR = NP // 16


def _deg_body(dst_hbm, out_hbm, dbuf, ones_v, zbuf, acc):
    c = lax.axis_index("c")
    s = lax.axis_index("s")
    zero16 = jnp.zeros((16,), jnp.float32)
    one16 = jnp.ones((16,), jnp.float32)
    for k in range(CH // 16):
        ones_v[pl.ds(16 * k, 16)] = one16

    def zb(i, _):
        zbuf[pl.ds(i * 16, 16)] = zero16
        return 0

    lax.fori_loop(0, 196, zb, 0)
    pltpu.sync_copy(zbuf.at[pl.ds(0, DEG_TPR)], acc.at[pl.ds(s * DEG_TPR, DEG_TPR)])
    plsc.subcore_barrier()

    def body(k, _):
        j = c * (EC // 2) + s + 16 * k

        @pl.when(s + 16 * k < EC // 2)
        def _():
            pltpu.sync_copy(dst_hbm.at[j], dbuf)
            pltpu.sync_copy(ones_v, acc.at[dbuf], add=True)

        return 0

    lax.fori_loop(0, (EC // 2 + 15) // 16, body, 0)
    plsc.subcore_barrier()
    pltpu.sync_copy(acc.at[pl.ds(s * DEG_TPR, DEG_TPR)], zbuf.at[pl.ds(0, DEG_TPR)])
    pltpu.sync_copy(zbuf.at[pl.ds(0, DEG_TPR)],
                    out_hbm.at[pl.ds(c * NP + s * DEG_TPR, DEG_TPR)])


def _agg_body(g_hbm, src_hbm, dst_hbm, out_hbm, sbuf, dbuf, dloc, rows, zrows, acc, sem):
    c = lax.axis_index("c")
    s = lax.axis_index("s")
    base = c * HALF
    zero16 = jnp.zeros((16,), jnp.float32)

    def zb(i, _):
        for k in range(4):
            zrows[i, pl.ds(16 * k, 16)] = zero16
        return 0

    lax.fori_loop(0, CH, zb, 0)
    for k in range(12):
        pltpu.sync_copy(zrows, acc.at[pl.ds(s * TPR + k * 128, 128)])
    pltpu.sync_copy(zrows.at[pl.ds(0, TPR - 12 * 128)],
                    acc.at[pl.ds(s * TPR + 12 * 128, TPR - 12 * 128)])

    @pl.when(s == 15)
    def _():
        pltpu.sync_copy(zrows.at[pl.ds(0, 64)], acc.at[pl.ds(HALF, 64)])

    plsc.subcore_barrier()

    def body(k, _):
        j = s + 16 * k

        @pl.when(j < EC)
        def _():
            pltpu.sync_copy(src_hbm.at[j], sbuf)
            pltpu.sync_copy(dst_hbm.at[j], dbuf)
            cp = pltpu.async_copy(g_hbm.at[sbuf], rows, sem)
            for t in range(CH // 16):
                d16 = dbuf[pl.ds(16 * t, 16)]
                m = (d16 >= base) & (d16 < base + HALF)
                dloc[pl.ds(16 * t, 16)] = jnp.where(m, d16 - base, HALF)
            cp.wait()
            pltpu.sync_copy(rows, acc.at[dloc], add=True)

        return 0

    lax.fori_loop(0, (EC + 15) // 16, body, 0)
    plsc.subcore_barrier()

    def drain(k, _):
        m = s + 16 * k

        @pl.when(m < HALF // 128)
        def _():
            pltpu.sync_copy(acc.at[pl.ds(m * 128, 128)], rows)
            pltpu.sync_copy(rows, out_hbm.at[pl.ds(c * HALF + m * 128, 128)])

        @pl.when(m == HALF // 128)
        def _():
            pltpu.sync_copy(acc.at[pl.ds(m * 128, 64)], rows.at[pl.ds(0, 64)])
            pltpu.sync_copy(rows.at[pl.ds(0, 64)],
                            out_hbm.at[pl.ds(c * HALF + m * 128, 64)])

        return 0

    lax.fori_loop(0, (HALF // 128 + 16) // 16, drain, 0)


_SC_MESH = plsc.VectorSubcoreMesh(core_axis_name="c", subcore_axis_name="s")
_SC_PARAMS = pltpu.CompilerParams(use_tc_tiling_on_sc=False)

_deg_call = pl.kernel(
    _deg_body,
    out_type=jax.ShapeDtypeStruct((2 * NP,), jnp.float32),
    mesh=_SC_MESH,
    scratch_types=[
        pltpu.VMEM((CH,), jnp.int32),
        pltpu.VMEM((CH,), jnp.float32),
        pltpu.VMEM((3136,), jnp.float32),
        pltpu.VMEM_SHARED((NP,), jnp.float32),
    ],
    compiler_params=_SC_PARAMS,
)

_agg_call = pl.kernel(
    _agg_body,
    out_type=jax.ShapeDtypeStruct((NP, H), jnp.float32),
    mesh=_SC_MESH,
    scratch_types=[
        pltpu.VMEM((CH,), jnp.int32),
        pltpu.VMEM((CH,), jnp.int32),
        pltpu.VMEM((CH,), jnp.int32),
        pltpu.VMEM((CH, H), jnp.float32),
        pltpu.VMEM((CH, H), jnp.float32),
        pltpu.VMEM_SHARED((ACC_ROWS, H), jnp.float32),
        pltpu.SemaphoreType.DMA,
    ],
    compiler_params=_SC_PARAMS,
)


def kernel(x, edge_index, batch, W_in, b_in, conv_W, conv_b, gn_w, gn_b, gn_ms,
           Wc1, bc1, Wc2, bc2, Wc3, bc3, Ws1, bs1, Ws2, bs2):
    src2 = edge_index[0].reshape(EC, CH)
    dst2 = edge_index[1].reshape(EC, CH)

    degpart = _deg_call(dst2)
    deg = degpart[:N] + degpart[NP:NP + N] + 1.0
    dinv = lax.rsqrt(deg)

    h = x @ W_in + b_in
    for l in range(3):
        hlin = h @ conv_W[l]
        g = dinv[:, None] * hlin
        gp = jnp.pad(g, ((0, NP - N), (0, 0)))
        agg = _agg_call(gp, src2, dst2)[:N]
        t = dinv[:, None] * (agg + g) + conv_b[l]
        mean = jnp.mean(t, axis=0)
        sub = t - mean * gn_ms[l]
        var = jnp.mean(sub * sub, axis=0)
        t = gn_w[l] * sub / jnp.sqrt(var + 1e-5) + gn_b[l]
        h = h + jax.nn.relu(t)

    graph_emb = jnp.mean(h, axis=0, keepdims=True)
    z = jax.nn.relu(graph_emb @ Wc1 + bc1)
    z = jax.nn.relu(z @ Wc2 + bc2)
    logits = z @ Wc3 + bc3
    probs = jax.nn.softmax(logits, axis=-1)
    ns = jax.nn.relu(h @ Ws1 + bs1)
    node_hub_scores = jax.nn.sigmoid(ns @ Ws2 + bs2)[:, 0]
    return logits, probs, graph_emb, h, node_hub_scores
